# Initial kernel scaffold; baseline (speedup 1.0000x reference)
import jax
import jax.numpy as jnp
from jax.experimental import pallas as pl


def kernel(images, psf_params, n_sources, locs, fluxes, vertical_shift, horizontal_shift):
    raise NotImplementedError("write your pallas kernel here")


# speedup vs baseline: 24.7627x; 24.7627x over previous
import jax
import jax.numpy as jnp
from jax import lax
from jax.experimental import pallas as pl
from jax.experimental.pallas import tpu as pltpu

_C, _H, _W = 5, 512, 512
_NT = 1024
_B = 128


def _img_body(sh_ref, img_ref, out_ref):
    v = sh_ref[0]
    h = sh_ref[1]
    x = img_ref[...]
    x = pltpu.roll(x, jnp.mod(v, _H), 1)
    x = pltpu.roll(x, jnp.mod(h, _W), 2)
    out_ref[...] = x


def _cat_body(sh_ref, np_ref, nc_ref, l0p_ref, l0c_ref, l1p_ref, l1c_ref,
              fp_ref, fc_ref, no_ref, l0o_ref, l1o_ref, fo_ref):
    vs = sh_ref[0]
    hs = sh_ref[1]
    dv = 2 * vs
    dh = 2 * hs
    rb = pl.program_id(0)
    gbase = rb * _B

    n = jnp.concatenate([np_ref[...], nc_ref[...]], axis=0)
    l0 = jnp.concatenate([l0p_ref[...], l0c_ref[...]], axis=0)
    l1 = jnp.concatenate([l1p_ref[...], l1c_ref[...]], axis=0)
    f = jnp.concatenate([fp_ref[...], fc_ref[...]], axis=0)

    gi = lax.broadcasted_iota(jnp.int32, (2 * _B, _NT), 0) + (gbase - _B)
    gj = lax.broadcasted_iota(jnp.int32, (2 * _B, _NT), 1)

    onf = (n > 0).astype(jnp.float32)
    half = jnp.float32(0.5)
    two = jnp.float32(2.0)
    u0 = ((gi.astype(jnp.float32) + l0) * half + vs.astype(jnp.float32)) * two
    u1 = ((gj.astype(jnp.float32) + l1) * half + hs.astype(jnp.float32)) * two
    t0 = jnp.floor(u0)
    t1 = jnp.floor(u1)
    nl0 = (u0 - t0) * onf
    nl1 = (u1 - t1) * onf
    rf0 = t0.astype(jnp.int32) - gi - dv
    rf1 = t1.astype(jnp.int32) - gj - dh
    vf = f * onf

    di = lax.broadcasted_iota(jnp.int32, (_B, _NT), 0) + gbase
    dj = lax.broadcasted_iota(jnp.int32, (_B, _NT), 1)

    acc_n = jnp.zeros((_B, _NT), jnp.float32)
    acc_l0 = jnp.zeros((_B, _NT), jnp.float32)
    acc_l1 = jnp.zeros((_B, _NT), jnp.float32)
    acc_f = jnp.zeros((_B, _NT), jnp.float32)
    for r0 in (0, 1):
        for r1 in (0, 1):
            m = ((rf0 == r0) & (rf1 == r1)).astype(jnp.float32) * onf
            csh = jnp.mod(dh + r1, _NT)
            rsh = jnp.mod(dv + r0 - _B, 2 * _B)
            valid = ((di >= dv + r0) & (di < _NT + dv + r0) &
                     (dj >= dh + r1) & (dj < _NT + dh + r1))
            vmask = valid.astype(jnp.float32)

            def shift(x, csh=csh, rsh=rsh, vmask=vmask):
                y = pltpu.roll(x, csh, 1)
                y = pltpu.roll(y, rsh, 0)[:_B]
                return y * vmask

            acc_n = acc_n + shift(m)
            acc_l0 = acc_l0 + shift(nl0 * m)
            acc_l1 = acc_l1 + shift(nl1 * m)
            acc_f = acc_f + shift(vf * m)

    no_ref[...] = jnp.minimum(acc_n, 1.0).astype(jnp.int32)
    l0o_ref[...] = acc_l0
    l1o_ref[...] = acc_l1
    fo_ref[...] = acc_f


def kernel(images, psf_params, n_sources, locs, fluxes, vertical_shift,
           horizontal_shift):
    sh = jnp.stack([jnp.asarray(vertical_shift, jnp.int32),
                    jnp.asarray(horizontal_shift, jnp.int32)])
    smem = pl.BlockSpec(memory_space=pltpu.SMEM)

    img = pl.pallas_call(
        _img_body,
        out_shape=jax.ShapeDtypeStruct((_C, _H, _W), jnp.float32),
        in_specs=[smem, pl.BlockSpec((_C, _H, _W), lambda: (0, 0, 0))],
        out_specs=pl.BlockSpec((_C, _H, _W), lambda: (0, 0, 0)),
    )(sh, images)

    l0 = locs[:, :, 0, 0]
    l1 = locs[:, :, 0, 1]
    f2 = fluxes.reshape(_NT, _NT)

    def prev_map(rb):
        return (jnp.maximum(rb - 1, 0), 0)

    def cur_map(rb):
        return (rb, 0)

    bs_prev = pl.BlockSpec((_B, _NT), prev_map)
    bs_cur = pl.BlockSpec((_B, _NT), cur_map)

    n_out, lo0, lo1, f_out = pl.pallas_call(
        _cat_body,
        grid=(_NT // _B,),
        out_shape=(jax.ShapeDtypeStruct((_NT, _NT), jnp.int32),
                   jax.ShapeDtypeStruct((_NT, _NT), jnp.float32),
                   jax.ShapeDtypeStruct((_NT, _NT), jnp.float32),
                   jax.ShapeDtypeStruct((_NT, _NT), jnp.float32)),
        in_specs=[smem,
                  bs_prev, bs_cur, bs_prev, bs_cur,
                  bs_prev, bs_cur, bs_prev, bs_cur],
        out_specs=(bs_cur, bs_cur, bs_cur, bs_cur),
    )(sh, n_sources, n_sources, l0, l0, l1, l1, f2, f2)

    locs_out = jnp.stack([lo0, lo1], axis=-1).reshape(_NT, _NT, 1, 2)
    return (img, psf_params, locs_out, f_out.reshape(_NT, _NT, 1, 1), n_out)


# speedup vs baseline: 49.6336x; 2.0044x over previous
import jax
import jax.numpy as jnp
from jax import lax
from jax.experimental import pallas as pl
from jax.experimental.pallas import tpu as pltpu

_C, _H, _W = 5, 512, 512
_NT = 1024
_B = 128
_BH = 32


def _img_body(sh_ref, img_ref, out_ref):
    v = sh_ref[0]
    h = sh_ref[1]
    x = img_ref[...]
    x = pltpu.roll(x, jnp.mod(v, _H), 1)
    x = pltpu.roll(x, jnp.mod(h, _W), 2)
    out_ref[...] = x


def _cat_body(sh_ref, nh_ref, nc_ref, l0h_ref, l0c_ref, l1h_ref, l1c_ref,
              fh_ref, fc_ref, no_ref, l0o_ref, l1o_ref, fo_ref):
    vs = sh_ref[0]
    hs = sh_ref[1]
    dv = 2 * vs
    dh = 2 * hs
    rb = pl.program_id(0)
    gbase = rb * _B
    wrows = _BH + _B

    n = jnp.concatenate([nh_ref[...], nc_ref[...]], axis=0)
    l0 = jnp.concatenate([l0h_ref[...], l0c_ref[...]], axis=0)
    l1 = jnp.concatenate([l1h_ref[...], l1c_ref[...]], axis=0)
    f = jnp.concatenate([fh_ref[...], fc_ref[...]], axis=0)

    gi = lax.broadcasted_iota(jnp.int32, (wrows, _NT), 0) + (gbase - _BH)
    gj = lax.broadcasted_iota(jnp.int32, (wrows, _NT), 1)

    onf = (n > 0).astype(jnp.float32)
    half = jnp.float32(0.5)
    two = jnp.float32(2.0)
    u0 = ((gi.astype(jnp.float32) + l0) * half + vs.astype(jnp.float32)) * two
    u1 = ((gj.astype(jnp.float32) + l1) * half + hs.astype(jnp.float32)) * two
    t0 = jnp.floor(u0)
    t1 = jnp.floor(u1)
    nl0 = (u0 - t0) * onf
    nl1 = (u1 - t1) * onf
    rf0 = (t0.astype(jnp.int32) - gi - dv).astype(jnp.float32)
    rf1 = (t1.astype(jnp.int32) - gj - dh).astype(jnp.float32)
    vf = f * onf

    csh = jnp.mod(dh, _NT)
    rsh = jnp.mod(dv - _BH, wrows)

    def prep(x):
        z = pltpu.roll(pltpu.roll(x, csh, 1), rsh, 0)
        return z, pltpu.roll(z, 1, 0)

    z_cnt = prep(onf)
    z_l0 = prep(nl0)
    z_l1 = prep(nl1)
    z_f = prep(vf)
    z_a = prep(rf0)
    z_b = prep(rf1)

    di = lax.broadcasted_iota(jnp.int32, (_B, _NT), 0) + gbase
    dj = lax.broadcasted_iota(jnp.int32, (_B, _NT), 1)

    acc_n = jnp.zeros((_B, _NT), jnp.float32)
    acc_l0 = jnp.zeros((_B, _NT), jnp.float32)
    acc_l1 = jnp.zeros((_B, _NT), jnp.float32)
    acc_f = jnp.zeros((_B, _NT), jnp.float32)
    for r0 in (0, 1):
        for r1 in (0, 1):
            def rd(zp, r0=r0, r1=r1):
                y = zp[r0]
                if r1:
                    y = pltpu.roll(y, 1, 1)
                return y[:_B]

            a = rd(z_a)
            b = rd(z_b)
            am = a if r0 else (1.0 - a)
            bm = b if r1 else (1.0 - b)
            valid = ((di >= dv + r0) & (di < _NT + dv + r0) &
                     (dj >= dh + r1) & (dj < _NT + dh + r1))
            m = am * bm * valid.astype(jnp.float32)
            acc_n = acc_n + rd(z_cnt) * m
            acc_l0 = acc_l0 + rd(z_l0) * m
            acc_l1 = acc_l1 + rd(z_l1) * m
            acc_f = acc_f + rd(z_f) * m

    no_ref[...] = jnp.minimum(acc_n, 1.0).astype(jnp.int32)
    l0o_ref[...] = acc_l0
    l1o_ref[...] = acc_l1
    fo_ref[...] = acc_f


def kernel(images, psf_params, n_sources, locs, fluxes, vertical_shift,
           horizontal_shift):
    sh = jnp.stack([jnp.asarray(vertical_shift, jnp.int32),
                    jnp.asarray(horizontal_shift, jnp.int32)])
    smem = pl.BlockSpec(memory_space=pltpu.SMEM)

    img = pl.pallas_call(
        _img_body,
        out_shape=jax.ShapeDtypeStruct((_C, _H, _W), jnp.float32),
        in_specs=[smem, pl.BlockSpec((_C, _H, _W), lambda: (0, 0, 0))],
        out_specs=pl.BlockSpec((_C, _H, _W), lambda: (0, 0, 0)),
    )(sh, images)

    l0 = locs[:, :, 0, 0]
    l1 = locs[:, :, 0, 1]
    f2 = fluxes.reshape(_NT, _NT)

    def halo_map(rb):
        return (jnp.maximum(rb * (_B // _BH) - 1, 0), 0)

    def cur_map(rb):
        return (rb, 0)

    bs_halo = pl.BlockSpec((_BH, _NT), halo_map)
    bs_cur = pl.BlockSpec((_B, _NT), cur_map)

    n_out, lo0, lo1, f_out = pl.pallas_call(
        _cat_body,
        grid=(_NT // _B,),
        out_shape=(jax.ShapeDtypeStruct((_NT, _NT), jnp.int32),
                   jax.ShapeDtypeStruct((_NT, _NT), jnp.float32),
                   jax.ShapeDtypeStruct((_NT, _NT), jnp.float32),
                   jax.ShapeDtypeStruct((_NT, _NT), jnp.float32)),
        in_specs=[smem,
                  bs_halo, bs_cur, bs_halo, bs_cur,
                  bs_halo, bs_cur, bs_halo, bs_cur],
        out_specs=(bs_cur, bs_cur, bs_cur, bs_cur),
    )(sh, n_sources, n_sources, l0, l0, l1, l1, f2, f2)

    locs_out = jnp.stack([lo0, lo1], axis=-1).reshape(_NT, _NT, 1, 2)
    return (img, psf_params, locs_out, f_out.reshape(_NT, _NT, 1, 1), n_out)
